# baseline (speedup 1.0000x reference)
import functools

import jax
import jax.numpy as jnp
from jax import lax
from jax.experimental import pallas as pl

HIDDEN = 4096
N_EXPERTS = 64
TOP_K = 8
ROW_BLOCK = 512


def _router_body(x_ref, wt_ref, idx_ref, w_ref, p_ref):
    x_blk = x_ref[...]
    w = wt_ref[...]
    logits_t = lax.dot_general(
        w, x_blk, (((1,), (1,)), ((), ())),
        preferred_element_type=jnp.float32)
    logits = logits_t.T

    m = jnp.max(logits, axis=1, keepdims=True)
    e = jnp.exp(logits - m)
    probs = e / jnp.sum(e, axis=1, keepdims=True)
    p_ref[...] = probs

    iota = lax.broadcasted_iota(jnp.int32, probs.shape, 1)
    vals = probs
    wt_cols = []
    idx_cols = []
    for _ in range(TOP_K):
        mx = jnp.max(vals, axis=1, keepdims=True)
        cand = jnp.where(vals == mx, iota, N_EXPERTS)
        amin = jnp.min(cand, axis=1, keepdims=True)
        wt_cols.append(mx)
        idx_cols.append(amin)
        vals = jnp.where(iota == amin, -jnp.inf, vals)

    weights = jnp.concatenate(wt_cols, axis=1)
    weights = weights / (jnp.sum(weights, axis=1, keepdims=True) + 1e-9)
    idx_ref[...] = jnp.concatenate(idx_cols, axis=1)
    w_ref[...] = weights


@jax.jit
def _router(flat, w_t):
    n_rows = flat.shape[0]
    grid = (n_rows // ROW_BLOCK,)
    return pl.pallas_call(
        _router_body,
        grid=grid,
        in_specs=[
            pl.BlockSpec((ROW_BLOCK, HIDDEN), lambda i: (i, 0)),
            pl.BlockSpec((N_EXPERTS, HIDDEN), lambda i: (0, 0)),
        ],
        out_specs=[
            pl.BlockSpec((ROW_BLOCK, TOP_K), lambda i: (i, 0)),
            pl.BlockSpec((ROW_BLOCK, TOP_K), lambda i: (i, 0)),
            pl.BlockSpec((ROW_BLOCK, N_EXPERTS), lambda i: (i, 0)),
        ],
        out_shape=[
            jax.ShapeDtypeStruct((n_rows, TOP_K), jnp.int32),
            jax.ShapeDtypeStruct((n_rows, TOP_K), jnp.float32),
            jax.ShapeDtypeStruct((n_rows, N_EXPERTS), jnp.float32),
        ],
    )(flat, w_t)


def kernel(x, W):
    flat = x.reshape(-1, x.shape[-1])
    indices, weights, probs = _router(flat, W)
    return indices, weights.astype(x.dtype), probs


# speedup vs baseline: 1.3136x; 1.3136x over previous
import functools

import jax
import jax.numpy as jnp
from jax import lax
from jax.experimental import pallas as pl

HIDDEN = 4096
N_EXPERTS = 64
TOP_K = 8
ROW_BLOCK = 512


def _router_body(x_ref, wt_ref, idx_ref, w_ref, p_ref):
    x_blk = x_ref[...]
    w = wt_ref[...]
    logits_t = lax.dot_general(
        w, x_blk, (((1,), (1,)), ((), ())),
        preferred_element_type=jnp.float32)

    m = jnp.max(logits_t, axis=0, keepdims=True)
    e = jnp.exp(logits_t - m)
    probs_t = e / jnp.sum(e, axis=0, keepdims=True)
    p_ref[...] = probs_t.T

    iota = lax.broadcasted_iota(jnp.int32, probs_t.shape, 0)
    vals = probs_t
    wt_rows = []
    idx_rows = []
    for _ in range(TOP_K):
        mx = jnp.max(vals, axis=0, keepdims=True)
        cand = jnp.where(vals == mx, iota, N_EXPERTS)
        amin = jnp.min(cand, axis=0, keepdims=True)
        wt_rows.append(mx)
        idx_rows.append(amin)
        vals = jnp.where(iota == amin, -jnp.inf, vals)

    weights_t = jnp.concatenate(wt_rows, axis=0)
    weights_t = weights_t / (jnp.sum(weights_t, axis=0, keepdims=True) + 1e-9)
    idx_ref[...] = jnp.concatenate(idx_rows, axis=0).T
    w_ref[...] = weights_t.T


@jax.jit
def _router(flat, w_t):
    n_rows = flat.shape[0]
    grid = (n_rows // ROW_BLOCK,)
    return pl.pallas_call(
        _router_body,
        grid=grid,
        in_specs=[
            pl.BlockSpec((ROW_BLOCK, HIDDEN), lambda i: (i, 0)),
            pl.BlockSpec((N_EXPERTS, HIDDEN), lambda i: (0, 0)),
        ],
        out_specs=[
            pl.BlockSpec((ROW_BLOCK, TOP_K), lambda i: (i, 0)),
            pl.BlockSpec((ROW_BLOCK, TOP_K), lambda i: (i, 0)),
            pl.BlockSpec((ROW_BLOCK, N_EXPERTS), lambda i: (i, 0)),
        ],
        out_shape=[
            jax.ShapeDtypeStruct((n_rows, TOP_K), jnp.int32),
            jax.ShapeDtypeStruct((n_rows, TOP_K), jnp.float32),
            jax.ShapeDtypeStruct((n_rows, N_EXPERTS), jnp.float32),
        ],
    )(flat, w_t)


def kernel(x, W):
    flat = x.reshape(-1, x.shape[-1])
    indices, weights, probs = _router(flat, W)
    return indices, weights.astype(x.dtype), probs


# speedup vs baseline: 1.3751x; 1.0468x over previous
import functools

import jax
import jax.numpy as jnp
from jax import lax
from jax.experimental import pallas as pl

HIDDEN = 4096
N_EXPERTS = 64
TOP_K = 8
ROW_BLOCK = 1024


def _router_body(x_ref, wt_ref, idx_ref, w_ref, p_ref):
    x_blk = x_ref[...]
    w = wt_ref[...]
    logits_t = lax.dot_general(
        w, x_blk, (((1,), (1,)), ((), ())),
        preferred_element_type=jnp.float32)

    m = jnp.max(logits_t, axis=0, keepdims=True)
    e = jnp.exp(logits_t - m)
    probs_t = e / jnp.sum(e, axis=0, keepdims=True)
    p_ref[...] = probs_t.T

    iota = lax.broadcasted_iota(jnp.int32, probs_t.shape, 0)
    vals = probs_t
    wt_rows = []
    idx_rows = []
    for _ in range(TOP_K):
        mx = jnp.max(vals, axis=0, keepdims=True)
        cand = jnp.where(vals == mx, iota, N_EXPERTS)
        amin = jnp.min(cand, axis=0, keepdims=True)
        wt_rows.append(mx)
        idx_rows.append(amin)
        vals = jnp.where(iota == amin, -jnp.inf, vals)

    weights_t = jnp.concatenate(wt_rows, axis=0)
    weights_t = weights_t / (jnp.sum(weights_t, axis=0, keepdims=True) + 1e-9)
    idx_ref[...] = jnp.concatenate(idx_rows, axis=0).T
    w_ref[...] = weights_t.T


@jax.jit
def _router(flat, w_t):
    n_rows = flat.shape[0]
    grid = (n_rows // ROW_BLOCK,)
    return pl.pallas_call(
        _router_body,
        grid=grid,
        in_specs=[
            pl.BlockSpec((ROW_BLOCK, HIDDEN), lambda i: (i, 0)),
            pl.BlockSpec((N_EXPERTS, HIDDEN), lambda i: (0, 0)),
        ],
        out_specs=[
            pl.BlockSpec((ROW_BLOCK, TOP_K), lambda i: (i, 0)),
            pl.BlockSpec((ROW_BLOCK, TOP_K), lambda i: (i, 0)),
            pl.BlockSpec((ROW_BLOCK, N_EXPERTS), lambda i: (i, 0)),
        ],
        out_shape=[
            jax.ShapeDtypeStruct((n_rows, TOP_K), jnp.int32),
            jax.ShapeDtypeStruct((n_rows, TOP_K), jnp.float32),
            jax.ShapeDtypeStruct((n_rows, N_EXPERTS), jnp.float32),
        ],
    )(flat, w_t)


def kernel(x, W):
    flat = x.reshape(-1, x.shape[-1])
    indices, weights, probs = _router(flat, W)
    return indices, weights.astype(x.dtype), probs
